# baseline (speedup 1.0000x reference)
import jax
import jax.numpy as jnp
from jax.experimental import pallas as pl

_RADIUS2 = 64.0
_KT = 8


def _dg0(w, x):
    return jax.lax.dot_general(w, x, (((0,), (0,)), ((), ())),
                               preferred_element_type=jnp.float32)


def _body(pos_ref, feat_ref, gf_ref, wg_ref, bg_ref, wax_ref, wbx_ref,
          wl_ref, bl_ref, gout_ref, outp_ref, outf_ref):
    p = 3
    n = pos_ref.shape[2]
    g2 = wg_ref.shape[1]
    g = g2 // 2
    pf = jnp.concatenate(
        [pos_ref[0], jnp.ones((1, n), jnp.float32), feat_ref[0]],
        axis=0)
    nTx = pf.T
    aggs = jnp.max(pf, axis=1, keepdims=True)
    agg_row = jnp.concatenate([aggs[:p].T, aggs[p + 1:].T], axis=1)
    g_lin = (jnp.dot(agg_row, wg_ref[:32, :], preferred_element_type=jnp.float32)
             + jnp.dot(gf_ref[0], wg_ref[32:, :], preferred_element_type=jnp.float32)
             + bg_ref[...])
    g_out = jnp.tanh(g_lin)
    gout_ref[0] = g_out[:, :g]
    gterm = (jnp.dot(g_out[:, g:], wl_ref[64:, :], preferred_element_type=jnp.float32)
             + bl_ref[...])
    gcol = gterm.T
    ts = n // _KT
    xr = pf[0:1, :]
    yr = pf[1:2, :]
    zr = pf[2:3, :]
    acc = [None] * _KT

    def _add(a, b):
        return b if a is None else a + b

    for r in range(_KT):
        rs = slice(r * ts, (r + 1) * ts)
        xall = nTx[rs, 0:1]
        yall = nTx[rs, 1:2]
        zall = nTx[rs, 2:3]
        for s in range(r + 1):
            cs = slice(s * ts, (s + 1) * ts)
            dx = xall - xr[:, cs]
            dy = yall - yr[:, cs]
            dz = zall - zr[:, cs]
            d2 = dx * dx + dy * dy + dz * dz
            mb = (d2 < _RADIUS2).astype(jnp.float32)
            acc[s] = _add(acc[s], jnp.dot(pf[:, rs], mb,
                                          preferred_element_type=jnp.float32))
            if r != s:
                tall = jnp.dot(mb, nTx[cs, :], preferred_element_type=jnp.float32)
                acc[r] = _add(acc[r], tall.T)
    sums = jnp.concatenate(acc, axis=1)
    cnt = jnp.maximum(sums[p:p + 1, :], 1.0)
    nmall = sums / cnt
    lin = (_dg0(wax_ref[...], pf) + _dg0(wbx_ref[...], nmall) + gcol)
    t_all = jnp.tanh(lin)
    outp_ref[0] = t_all[:p]
    outf_ref[0] = t_all[p:]


def kernel(positions, features, global_features, W_g, b_g, W_l, b_l):
    B, P, N = positions.shape
    F = features.shape[1]
    G = global_features.shape[1]
    C = P + F
    G2 = 2 * G

    zrow = jnp.zeros((1, C), jnp.float32)
    wax = jnp.concatenate([W_l[:P], zrow, W_l[P:C]], axis=0)
    wbx = jnp.concatenate([W_l[C:C + P], zrow, W_l[C + P:2 * C]], axis=0)
    gf3 = global_features.reshape(B, 1, G)
    bg2 = b_g.reshape(1, G2)
    bl2 = b_l.reshape(1, C)

    ws = lambda a, b: pl.BlockSpec((a, b), lambda i: (0, 0))
    g_out3, positions_new, features_new = pl.pallas_call(
        _body,
        grid=(B,),
        in_specs=[
            pl.BlockSpec((1, P, N), lambda i: (i, 0, 0)),
            pl.BlockSpec((1, F, N), lambda i: (i, 0, 0)),
            pl.BlockSpec((1, 1, G), lambda i: (i, 0, 0)),
            ws(C + G, G2), ws(1, G2), ws(C + 1, C), ws(C + 1, C),
            ws(2 * C + G, C), ws(1, C),
        ],
        out_specs=(
            pl.BlockSpec((1, 1, G), lambda i: (i, 0, 0)),
            pl.BlockSpec((1, P, N), lambda i: (i, 0, 0)),
            pl.BlockSpec((1, F, N), lambda i: (i, 0, 0)),
        ),
        out_shape=(
            jax.ShapeDtypeStruct((B, 1, G), jnp.float32),
            jax.ShapeDtypeStruct((B, P, N), jnp.float32),
            jax.ShapeDtypeStruct((B, F, N), jnp.float32),
        ),
    )(positions, features, gf3, W_g, bg2, wax, wbx, W_l, bl2)

    global_new = g_out3.reshape(B, G)
    return (positions_new, features_new, global_new)
